# baseline (speedup 1.0000x reference)
import jax
import jax.numpy as jnp
from jax import lax
from jax.experimental import pallas as pl
from jax.experimental.pallas import tpu as pltpu

D_MODEL = 2048
D_FF = 8192
NUM_OPS = 16

BM = 1024
BF = 1024
BN = 256
NF = D_FF // BF
NN = D_MODEL // BN
BR = 1024


def _route_cast_body(x_ref, xb_ref, op_ref):
    @pl.when(pl.program_id(0) == 0)
    def _route():
        v = x_ref[0:1, :NUM_OPS]
        mx = jnp.max(v, axis=1, keepdims=True)
        idx = lax.broadcasted_iota(jnp.int32, v.shape, 1)
        cand = jnp.where(v == mx, idx, NUM_OPS)
        op_ref[0] = jnp.min(cand)

    xb_ref[...] = x_ref[...].astype(jnp.bfloat16)


def _ffn_body(op_ref, x_ref, w1_ref, b1_ref, w2_ref, b2_ref, o_ref, h_ref):
    j = pl.program_id(1)

    @pl.when(j < NF)
    def _hidden():
        w1b = w1_ref[0].astype(jnp.bfloat16)
        h = jnp.dot(x_ref[...], w1b, preferred_element_type=jnp.float32)
        h = jnp.maximum(h + b1_ref[0], 0.0)
        h_ref[:, pl.ds(j * BF, BF)] = h.astype(jnp.bfloat16)

    @pl.when(j >= NF)
    def _output():
        w2b = w2_ref[0].astype(jnp.bfloat16)
        o_ref[...] = (
            jnp.dot(h_ref[...], w2b, preferred_element_type=jnp.float32)
            + b2_ref[0]
        )


def kernel(x, W1, b1, W2, b2):
    batch, seq, d_model = x.shape
    m_total = batch * seq
    xf = x.reshape(m_total, d_model)

    xb, op_arr = pl.pallas_call(
        _route_cast_body,
        grid=(m_total // BR,),
        in_specs=[pl.BlockSpec((BR, d_model), lambda i: (i, 0))],
        out_specs=[
            pl.BlockSpec((BR, d_model), lambda i: (i, 0)),
            pl.BlockSpec(memory_space=pltpu.SMEM),
        ],
        out_shape=[
            jax.ShapeDtypeStruct((m_total, d_model), jnp.bfloat16),
            jax.ShapeDtypeStruct((1,), jnp.int32),
        ],
        compiler_params=pltpu.CompilerParams(
            dimension_semantics=("arbitrary",),
        ),
    )(xf)

    b1r = b1.reshape(b1.shape[0], 1, D_FF)
    b2r = b2.reshape(b2.shape[0], 1, d_model)

    grid = (m_total // BM, NF + NN)

    out = pl.pallas_call(
        _ffn_body,
        grid_spec=pltpu.PrefetchScalarGridSpec(
            num_scalar_prefetch=1,
            grid=grid,
            in_specs=[
                pl.BlockSpec((BM, d_model), lambda m, j, op: (m, 0)),
                pl.BlockSpec(
                    (1, d_model, BF),
                    lambda m, j, op: (op[0], 0, jnp.minimum(j, NF - 1)),
                ),
                pl.BlockSpec(
                    (1, 1, BF),
                    lambda m, j, op: (op[0], 0, jnp.minimum(j, NF - 1)),
                ),
                pl.BlockSpec(
                    (1, D_FF, BN),
                    lambda m, j, op: (op[0], 0, jnp.maximum(j - NF, 0)),
                ),
                pl.BlockSpec(
                    (1, 1, BN),
                    lambda m, j, op: (op[0], 0, jnp.maximum(j - NF, 0)),
                ),
            ],
            out_specs=pl.BlockSpec(
                (BM, BN),
                lambda m, j, op: (m, jnp.maximum(j - NF, 0)),
            ),
            scratch_shapes=[pltpu.VMEM((BM, D_FF), jnp.bfloat16)],
        ),
        out_shape=jax.ShapeDtypeStruct((m_total, d_model), jnp.float32),
        compiler_params=pltpu.CompilerParams(
            dimension_semantics=("parallel", "arbitrary"),
            vmem_limit_bytes=64 * 1024 * 1024,
        ),
    )(op_arr, xb, W1, b1r, W2, b2r)

    return out.reshape(batch, seq, d_model)


# speedup vs baseline: 1.0429x; 1.0429x over previous
import jax
import jax.numpy as jnp
from jax import lax
from jax.experimental import pallas as pl
from jax.experimental.pallas import tpu as pltpu

D_MODEL = 2048
D_FF = 8192
NUM_OPS = 16

BM = 1024
BF = 2048
BN = 256
NF = D_FF // BF
NN = D_MODEL // BN


def _route_body(x_ref, op_ref):
    v = x_ref[...]
    mx = jnp.max(v, axis=1, keepdims=True)
    idx = lax.broadcasted_iota(jnp.int32, v.shape, 1)
    cand = jnp.where(v == mx, idx, NUM_OPS)
    op_ref[0] = jnp.min(cand)


def _hidden_body(op_ref, x_ref, w1_ref, b1_ref, h_ref):
    h = jnp.dot(x_ref[...], w1_ref[0], preferred_element_type=jnp.float32)
    h_ref[...] = jnp.maximum(h + b1_ref[0], 0.0).astype(jnp.bfloat16)


def _output_body(op_ref, h_ref, w2_ref, b2_ref, o_ref):
    w2b = w2_ref[0].astype(jnp.bfloat16)
    o_ref[...] = (
        jnp.dot(h_ref[...], w2b, preferred_element_type=jnp.float32)
        + b2_ref[0]
    )


def kernel(x, W1, b1, W2, b2):
    batch, seq, d_model = x.shape
    m_total = batch * seq
    xf = x.reshape(m_total, d_model)

    op_arr = pl.pallas_call(
        _route_body,
        out_shape=jax.ShapeDtypeStruct((1,), jnp.int32),
        out_specs=pl.BlockSpec(memory_space=pltpu.SMEM),
    )(xf[0:1, :NUM_OPS])

    b1r = b1.reshape(b1.shape[0], 1, D_FF)
    b2r = b2.reshape(b2.shape[0], 1, d_model)

    h = pl.pallas_call(
        _hidden_body,
        grid_spec=pltpu.PrefetchScalarGridSpec(
            num_scalar_prefetch=1,
            grid=(NF, m_total // BM),
            in_specs=[
                pl.BlockSpec((BM, d_model), lambda f, m, op: (m, 0)),
                pl.BlockSpec((1, d_model, BF), lambda f, m, op: (op[0], 0, f)),
                pl.BlockSpec((1, 1, BF), lambda f, m, op: (op[0], 0, f)),
            ],
            out_specs=pl.BlockSpec((BM, BF), lambda f, m, op: (m, f)),
        ),
        out_shape=jax.ShapeDtypeStruct((m_total, D_FF), jnp.bfloat16),
        compiler_params=pltpu.CompilerParams(
            dimension_semantics=("arbitrary", "arbitrary"),
        ),
    )(op_arr, xf, W1, b1r)

    out = pl.pallas_call(
        _output_body,
        grid_spec=pltpu.PrefetchScalarGridSpec(
            num_scalar_prefetch=1,
            grid=(m_total // BM, NN),
            in_specs=[
                pl.BlockSpec((BM, D_FF), lambda m, n, op: (m, 0)),
                pl.BlockSpec((1, D_FF, BN), lambda m, n, op: (op[0], 0, n)),
                pl.BlockSpec((1, 1, BN), lambda m, n, op: (op[0], 0, n)),
            ],
            out_specs=pl.BlockSpec((BM, BN), lambda m, n, op: (m, n)),
        ),
        out_shape=jax.ShapeDtypeStruct((m_total, d_model), jnp.float32),
        compiler_params=pltpu.CompilerParams(
            dimension_semantics=("arbitrary", "arbitrary"),
            vmem_limit_bytes=64 * 1024 * 1024,
        ),
    )(op_arr, h, W2, b2r)

    return out.reshape(batch, seq, d_model)


# speedup vs baseline: 1.0998x; 1.0546x over previous
import jax
import jax.numpy as jnp
from jax import lax
from jax.experimental import pallas as pl
from jax.experimental.pallas import tpu as pltpu

D_MODEL = 2048
D_FF = 8192
NUM_OPS = 16

BM = 1024
BF = 2048
BN = 512
NF = D_FF // BF
NN = D_MODEL // BN


def _route_body(x_ref, op_ref):
    v = x_ref[...]
    mx = jnp.max(v, axis=1, keepdims=True)
    idx = lax.broadcasted_iota(jnp.int32, v.shape, 1)
    cand = jnp.where(v == mx, idx, NUM_OPS)
    op_ref[0] = jnp.min(cand)


def _hidden_body(op_ref, x_ref, w1_ref, b1_ref, w2c_ref, h_ref, w2b_ref):
    h = jnp.dot(x_ref[...], w1_ref[0], preferred_element_type=jnp.float32)
    h_ref[...] = jnp.maximum(h + b1_ref[0], 0.0).astype(jnp.bfloat16)
    w2b_ref[...] = w2c_ref[0].astype(jnp.bfloat16)


def _output_body(op_ref, h_ref, w2_ref, b2_ref, o_ref):
    o_ref[...] = (
        jnp.dot(h_ref[...], w2_ref[...], preferred_element_type=jnp.float32)
        + b2_ref[0]
    )


def kernel(x, W1, b1, W2, b2):
    batch, seq, d_model = x.shape
    m_total = batch * seq
    xf = x.reshape(m_total, d_model)

    op_arr = pl.pallas_call(
        _route_body,
        out_shape=jax.ShapeDtypeStruct((1,), jnp.int32),
        out_specs=pl.BlockSpec(memory_space=pltpu.SMEM),
    )(xf[0:1, :NUM_OPS])

    b1r = b1.reshape(b1.shape[0], 1, D_FF)
    b2r = b2.reshape(b2.shape[0], 1, d_model)

    n_steps = NF * (m_total // BM)
    w2_rows = D_FF // n_steps
    h, w2b = pl.pallas_call(
        _hidden_body,
        grid_spec=pltpu.PrefetchScalarGridSpec(
            num_scalar_prefetch=1,
            grid=(NF, m_total // BM),
            in_specs=[
                pl.BlockSpec((BM, d_model), lambda f, m, op: (m, 0)),
                pl.BlockSpec((1, d_model, BF), lambda f, m, op: (op[0], 0, f)),
                pl.BlockSpec((1, 1, BF), lambda f, m, op: (op[0], 0, f)),
                pl.BlockSpec(
                    (1, w2_rows, d_model),
                    lambda f, m, op: (op[0], f * (m_total // BM) + m, 0),
                ),
            ],
            out_specs=[
                pl.BlockSpec((BM, BF), lambda f, m, op: (m, f)),
                pl.BlockSpec(
                    (w2_rows, d_model),
                    lambda f, m, op: (f * (m_total // BM) + m, 0),
                ),
            ],
        ),
        out_shape=[
            jax.ShapeDtypeStruct((m_total, D_FF), jnp.bfloat16),
            jax.ShapeDtypeStruct((D_FF, d_model), jnp.bfloat16),
        ],
        compiler_params=pltpu.CompilerParams(
            dimension_semantics=("arbitrary", "arbitrary"),
            vmem_limit_bytes=64 * 1024 * 1024,
        ),
    )(op_arr, xf, W1, b1r, W2)

    out = pl.pallas_call(
        _output_body,
        grid_spec=pltpu.PrefetchScalarGridSpec(
            num_scalar_prefetch=1,
            grid=(m_total // BM, NN),
            in_specs=[
                pl.BlockSpec((BM, D_FF), lambda m, n, op: (m, 0)),
                pl.BlockSpec((D_FF, BN), lambda m, n, op: (0, n)),
                pl.BlockSpec((1, 1, BN), lambda m, n, op: (op[0], 0, n)),
            ],
            out_specs=pl.BlockSpec((BM, BN), lambda m, n, op: (m, n)),
        ),
        out_shape=jax.ShapeDtypeStruct((m_total, d_model), jnp.float32),
        compiler_params=pltpu.CompilerParams(
            dimension_semantics=("arbitrary", "arbitrary"),
            vmem_limit_bytes=64 * 1024 * 1024,
        ),
    )(op_arr, h, w2b, b2r)

    return out.reshape(batch, seq, d_model)
